# baseline (speedup 1.0000x reference)
import functools

import jax
import jax.numpy as jnp
from jax import lax
from jax.experimental import pallas as pl
from jax.experimental.pallas import tpu as pltpu
from jax.experimental.pallas import tpu_sc as plsc

_VB = 8
_BL = 128


def _logits_t_body(w_ref, emb_ref, b_ref, out_ref):
    out_ref[...] = lax.dot_general(
        w_ref[...], emb_ref[...],
        (((1,), (1,)), ((), ())),
        preferred_element_type=jnp.float32,
    ) + b_ref[...]


def _build_logits_t(emb_table, fc_w, fc_b, vpad):
    vocab, dim = emb_table.shape
    w_pad = jnp.zeros((vpad, dim), jnp.float32).at[:vocab].set(fc_w)
    b_pad = jnp.zeros((vpad, 1), jnp.float32).at[:vocab, 0].set(fc_b)
    return pl.pallas_call(
        _logits_t_body,
        out_shape=jax.ShapeDtypeStruct((vpad, vocab), jnp.float32),
    )(w_pad, emb_table, b_pad)


def _make_gather(batch, ctx, vocab):
    info = plsc.get_sparse_core_info()
    nc, ns, nl = info.num_cores, info.num_subcores, info.num_lanes
    nw = nc * ns
    n_vb = vocab // _VB
    n_bt = batch // _BL
    kpv = _BL // nl
    mesh = plsc.VectorSubcoreMesh(core_axis_name="c", subcore_axis_name="s")

    @functools.partial(
        pl.kernel,
        mesh=mesh,
        compiler_params=pltpu.CompilerParams(use_tc_tiling_on_sc=False,
                                             needs_layout_passes=False),
        out_type=jax.ShapeDtypeStruct((ctx, n_vb, n_bt, _VB, _BL),
                                      jnp.float32),
        scratch_types=[
            pltpu.VMEM((batch * ctx,), jnp.int32),
            pltpu.VMEM((_VB, vocab), jnp.float32),
            pltpu.VMEM((n_bt, _VB, _BL), jnp.float32),
            pltpu.SemaphoreType.DMA,
        ],
    )
    def gather_k(mt_hbm, idx_hbm, out_hbm, idx_v, slab, tile, sem):
        wid = lax.axis_index("s") * nc + lax.axis_index("c")
        pltpu.sync_copy(idx_hbm, idx_v)
        nvb_mine = (n_vb + nw - 1 - wid) // nw

        def vb_body(k, carry):
            vb = wid + k * nw
            pltpu.sync_copy(mt_hbm.at[pl.ds(vb * _VB, _VB), :], slab)
            for c in range(ctx):

                def bt_body(j, carry2, c=c):
                    for q in range(kpv):
                        col_idx = idx_v[pl.ds(c * batch + (j * kpv + q) * nl,
                                              nl)]
                        for r in range(_VB):
                            row_idx = jnp.full((nl,), r, jnp.int32)
                            vals = plsc.load_gather(slab, [row_idx, col_idx])
                            tile[j, r, pl.ds(q * nl, nl)] = vals
                    return carry2

                lax.fori_loop(0, n_bt, bt_body, 0, unroll=2)
                pltpu.sync_copy(tile, out_hbm.at[c, vb])
            return carry

        lax.fori_loop(0, nvb_mine, vb_body, 0)

    return gather_k


def kernel(inputs, emb_table, fc_w, fc_b):
    batch, ctx = inputs.shape
    vocab = emb_table.shape[0]
    mt = _build_logits_t(emb_table, fc_w, fc_b, 1024)
    idx_t = inputs.astype(jnp.int32).T.reshape(ctx * batch)
    out5 = _make_gather(batch, ctx, vocab)(mt, idx_t)
    return out5.transpose((2, 4, 0, 1, 3)).reshape(batch, ctx, vocab)


# speedup vs baseline: 1.8296x; 1.8296x over previous
import functools

import jax
import jax.numpy as jnp
from jax import lax
from jax.experimental import pallas as pl
from jax.experimental.pallas import tpu as pltpu
from jax.experimental.pallas import tpu_sc as plsc

_VB = 8
_BL = 128


def _logits_t_body(w_ref, emb_ref, b_ref, out_ref):
    out_ref[...] = lax.dot_general(
        w_ref[...], emb_ref[...],
        (((1,), (1,)), ((), ())),
        preferred_element_type=jnp.float32,
    ) + b_ref[...]


def _build_logits_t(emb_table, fc_w, fc_b, vpad):
    vocab, dim = emb_table.shape
    w_pad = jnp.zeros((vpad, dim), jnp.float32).at[:vocab].set(fc_w)
    b_pad = jnp.zeros((vpad, 1), jnp.float32).at[:vocab, 0].set(fc_b)
    return pl.pallas_call(
        _logits_t_body,
        out_shape=jax.ShapeDtypeStruct((vpad, vocab), jnp.float32),
    )(w_pad, emb_table, b_pad)


def _make_gather(batch, ctx, vocab):
    info = plsc.get_sparse_core_info()
    nc, ns, nl = info.num_cores, info.num_subcores, info.num_lanes
    nw = nc * ns
    n_vb = vocab // _VB
    n_bt = batch // _BL
    kpv = _BL // nl
    mesh = plsc.VectorSubcoreMesh(core_axis_name="c", subcore_axis_name="s")

    @functools.partial(
        pl.kernel,
        mesh=mesh,
        compiler_params=pltpu.CompilerParams(use_tc_tiling_on_sc=False,
                                             needs_layout_passes=False),
        out_type=jax.ShapeDtypeStruct((ctx, n_vb, n_bt, _VB, _BL),
                                      jnp.float32),
        scratch_types=[
            pltpu.VMEM((batch * ctx,), jnp.int32),
            pltpu.VMEM((_VB, vocab), jnp.float32),
            pltpu.VMEM((n_bt, _VB, _BL), jnp.float32),
            pltpu.SemaphoreType.DMA,
        ],
    )
    def gather_k(mt_hbm, idx_hbm, out_hbm, idx_v, slab, tile, sem):
        wid = lax.axis_index("s") * nc + lax.axis_index("c")
        pltpu.sync_copy(idx_hbm, idx_v)
        nvb_mine = (n_vb + nw - 1 - wid) // nw

        def vb_body(k, carry):
            vb = wid + k * nw
            pltpu.sync_copy(mt_hbm.at[pl.ds(vb * _VB, _VB), :], slab)
            for c in range(ctx):

                def bt_body(j, carry2, c=c):
                    for q in range(kpv):
                        col_idx = idx_v[pl.ds(c * batch + (j * kpv + q) * nl,
                                              nl)]
                        vals = [
                            plsc.load_gather(
                                slab,
                                [jnp.full((nl,), r, jnp.int32), col_idx])
                            for r in range(_VB)
                        ]
                        for r in range(_VB):
                            tile[j, r, pl.ds(q * nl, nl)] = vals[r]
                    return carry2

                lax.fori_loop(0, n_bt, bt_body, 0, unroll=2)
                pltpu.sync_copy(tile, out_hbm.at[c, vb])
            return carry

        lax.fori_loop(0, nvb_mine, vb_body, 0)

    return gather_k


def kernel(inputs, emb_table, fc_w, fc_b):
    batch, ctx = inputs.shape
    vocab = emb_table.shape[0]
    mt = _build_logits_t(emb_table, fc_w, fc_b, 1024)
    idx_t = inputs.astype(jnp.int32).T.reshape(ctx * batch)
    out5 = _make_gather(batch, ctx, vocab)(mt, idx_t)
    return out5.transpose((2, 4, 0, 1, 3)).reshape(batch, ctx, vocab)
